# baseline (speedup 1.0000x reference)
import functools

import jax
import jax.numpy as jnp
from jax import lax
from jax.experimental import pallas as pl
from jax.experimental.pallas import tpu as pltpu
from jax.experimental.pallas import tpu_sc as plsc

_ROWS = 1000000
_COLS = 64
_NW = 32
_SPAN = 31248
_CHUNK = 504
_NCH = _SPAN // _CHUNK
_TAIL = _ROWS - _NW * _SPAN


def _sc_body(x_hbm, o_hbm, buf, in_sems, out_sems):
    c = lax.axis_index("c")
    s = lax.axis_index("s")
    wid = s * 2 + c
    base = wid * _SPAN

    def in_copy(k, b):
        return pltpu.make_async_copy(
            x_hbm.at[pl.ds(base + k * _CHUNK, _CHUNK), :],
            buf.at[b],
            in_sems.at[b],
        )

    def out_copy(k, b):
        return pltpu.make_async_copy(
            buf.at[b],
            o_hbm.at[pl.ds(base + k * _CHUNK, _CHUNK), :],
            out_sems.at[b],
        )

    for k in range(_NCH):
        b = k % 2
        if k >= 2:
            out_copy(k - 2, b).wait()
        in_copy(k, b).start()
        in_copy(k, b).wait()
        out_copy(k, b).start()
    out_copy(_NCH - 2, 0).wait()
    out_copy(_NCH - 1, 1).wait()

    @pl.when(wid == _NW - 1)
    def _():
        t0 = _NW * _SPAN
        pltpu.make_async_copy(
            x_hbm.at[pl.ds(t0, _TAIL), :],
            buf.at[0, pl.ds(0, _TAIL), :],
            in_sems.at[0],
        ).start()
        pltpu.make_async_copy(
            x_hbm.at[pl.ds(t0, _TAIL), :],
            buf.at[0, pl.ds(0, _TAIL), :],
            in_sems.at[0],
        ).wait()
        pltpu.make_async_copy(
            buf.at[0, pl.ds(0, _TAIL), :],
            o_hbm.at[pl.ds(t0, _TAIL), :],
            out_sems.at[0],
        ).start()
        pltpu.make_async_copy(
            buf.at[0, pl.ds(0, _TAIL), :],
            o_hbm.at[pl.ds(t0, _TAIL), :],
            out_sems.at[0],
        ).wait()


def kernel(table):
    mesh = plsc.VectorSubcoreMesh(core_axis_name="c", subcore_axis_name="s")
    f = functools.partial(
        pl.kernel,
        out_type=jax.ShapeDtypeStruct((_ROWS, _COLS), table.dtype),
        mesh=mesh,
        scratch_types=[
            pltpu.VMEM((2, _CHUNK, _COLS), table.dtype),
            pltpu.SemaphoreType.DMA((2,)),
            pltpu.SemaphoreType.DMA((2,)),
        ],
    )(_sc_body)
    return f(table)
